# Initial kernel scaffold; baseline (speedup 1.0000x reference)
import jax
import jax.numpy as jnp
from jax.experimental import pallas as pl


def kernel(x, batch):
    raise NotImplementedError("write your pallas kernel here")


# speedup vs baseline: 3.8632x; 3.8632x over previous
import functools

import jax
import jax.numpy as jnp
from jax import lax
from jax.experimental import pallas as pl
from jax.experimental.pallas import tpu as pltpu
from jax.experimental.pallas import tpu_sc as plsc

N_ROWS = 100000
D = 128
NSEG = 64
NLANE = 16
NVEC = D // NLANE

NWORKERS = 32
CHUNK = 256
NCHUNK = -(-N_ROWS // CHUNK)
LAST_BASE = N_ROWS - CHUNK
CHUNKS_PER_WORKER = -(-NCHUNK // NWORKERS)


def _sc_pass1_body(x_hbm, batch_hbm, sums_hbm, ssq_hbm, cnt_hbm,
                   xbuf, bbuf, acc, ssqacc, cnts):
  wid = lax.axis_index("s") * 2 + lax.axis_index("c")

  def _zero(r, carry):
    for j in range(NVEC):
      acc[r, pl.ds(j * NLANE, NLANE)] = jnp.zeros((NLANE,), jnp.float32)
    ssqacc[r] = jnp.zeros((NLANE,), jnp.float32)
    cnts[r] = jnp.zeros((NLANE,), jnp.int32)
    return carry
  lax.fori_loop(0, NSEG, _zero, 0)

  def _flush(prev, s, ssqv, run):
    for j in range(NVEC):
      sl = pl.ds(j * NLANE, NLANE)
      acc[prev, sl] = acc[prev, sl] + s[j]
    ssqacc[prev] = ssqacc[prev] + ssqv
    cnts[prev] = cnts[prev] + jnp.broadcast_to(run, (NLANE,))

  def process_chunk(c):
    base = jnp.minimum(c * CHUNK, LAST_BASE)
    skip_g = (c * CHUNK - base) // NLANE
    pltpu.sync_copy(x_hbm.at[pl.ds(base, CHUNK), :], xbuf)
    pltpu.sync_copy(batch_hbm.at[pl.ds(base, CHUNK)], bbuf)

    def group_body(g, carry):
      segv = bbuf[pl.ds(g * NLANE, NLANE)]
      for i in range(NLANE):
        s = list(carry[:NVEC])
        ssqv, run, prev = carry[NVEC], carry[NVEC + 1], carry[NVEC + 2]
        r = g * NLANE + i
        seg = segv[i]
        is_new = seg != prev

        @pl.when(is_new & (prev >= 0))
        def _do_flush():
          _flush(prev, s, ssqv, run)

        rowsq = jnp.zeros((NLANE,), jnp.float32)
        new_s = []
        for j in range(NVEC):
          xj = xbuf[r, pl.ds(j * NLANE, NLANE)]
          rowsq = rowsq + xj * xj
          new_s.append(jnp.where(is_new, xj, s[j] + xj))
        ssqv = jnp.where(is_new, rowsq, ssqv + rowsq)
        run = jnp.where(is_new, 1, run + 1)
        carry = tuple(new_s) + (ssqv, run, seg)
      return carry

    zeros = tuple(jnp.zeros((NLANE,), jnp.float32) for _ in range(NVEC))
    init = zeros + (jnp.zeros((NLANE,), jnp.float32),
                    jnp.int32(0), jnp.int32(-1))
    carry = lax.fori_loop(skip_g, CHUNK // NLANE, group_body, init)
    prev = carry[NVEC + 2]

    @pl.when(prev >= 0)
    def _final_flush():
      _flush(prev, list(carry[:NVEC]), carry[NVEC], carry[NVEC + 1])

  def chunk_body(i, carry):
    c = wid + i * NWORKERS

    @pl.when(c < NCHUNK)
    def _():
      process_chunk(c)
    return carry
  lax.fori_loop(0, CHUNKS_PER_WORKER, chunk_body, 0)

  pltpu.sync_copy(acc, sums_hbm.at[wid])
  pltpu.sync_copy(ssqacc, ssq_hbm.at[wid])
  pltpu.sync_copy(cnts, cnt_hbm.at[wid])


def _sc_pass1(x, batch):
  mesh = plsc.VectorSubcoreMesh(core_axis_name="c", subcore_axis_name="s",
                                num_cores=2, num_subcores=16)
  out_type = (
      jax.ShapeDtypeStruct((NWORKERS, NSEG, D), jnp.float32),
      jax.ShapeDtypeStruct((NWORKERS, NSEG, NLANE), jnp.float32),
      jax.ShapeDtypeStruct((NWORKERS, NSEG, NLANE), jnp.int32),
  )
  scratch = [
      pltpu.VMEM((CHUNK, D), jnp.float32),
      pltpu.VMEM((CHUNK,), jnp.int32),
      pltpu.VMEM((NSEG, D), jnp.float32),
      pltpu.VMEM((NSEG, NLANE), jnp.float32),
      pltpu.VMEM((NSEG, NLANE), jnp.int32),
  ]
  fn = pl.kernel(_sc_pass1_body, out_type=out_type, mesh=mesh,
                 scratch_types=scratch)
  return fn(x, batch)


def _combine_body(sums_ref, ssq_ref, cnt_ref, a_ref, b_ref):
  tot = jnp.sum(sums_ref[...], axis=0)
  cnt = (jnp.sum(cnt_ref[...], axis=(0, 2)) // NLANE).astype(jnp.float32)
  cntc = jnp.maximum(cnt, 1.0)
  mean = tot / cntc[:, None]
  ssq_tot = jnp.sum(ssq_ref[...], axis=(0, 2))
  sq_mean = jnp.maximum(ssq_tot / cntc - jnp.sum(mean * mean, axis=1), 0.0)
  rstd = jnp.where(sq_mean > 0, lax.rsqrt(sq_mean), 0.0)
  a_ref[...] = jnp.broadcast_to(rstd[:, None], (NSEG, D))
  b_ref[...] = mean * rstd[:, None]


def _tc_combine(sums, ssq, cnt):
  return pl.pallas_call(
      _combine_body,
      out_shape=(
          jax.ShapeDtypeStruct((NSEG, D), jnp.float32),
          jax.ShapeDtypeStruct((NSEG, D), jnp.float32),
      ),
  )(sums, ssq, cnt)


BLK = 1000
NBLK = N_ROWS // BLK


def _pass2_body(x_ref, b_ref, a_ref, mb_ref, o_ref):
  b = b_ref[0, 0, :]
  oh = (b[:, None] == lax.broadcasted_iota(jnp.int32, (BLK, NSEG), 1))
  oh = oh.astype(jnp.float32)
  rstd = jnp.dot(oh, a_ref[...], preferred_element_type=jnp.float32)
  mscaled = jnp.dot(oh, mb_ref[...], preferred_element_type=jnp.float32)
  o_ref[...] = x_ref[...] * rstd - mscaled


def _tc_pass2(x, batch, a, mb):
  batch3 = batch.reshape(NBLK, 1, BLK)
  return pl.pallas_call(
      _pass2_body,
      grid=(NBLK,),
      in_specs=[
          pl.BlockSpec((BLK, D), lambda i: (i, 0)),
          pl.BlockSpec((1, 1, BLK), lambda i: (i, 0, 0)),
          pl.BlockSpec((NSEG, D), lambda i: (0, 0)),
          pl.BlockSpec((NSEG, D), lambda i: (0, 0)),
      ],
      out_specs=pl.BlockSpec((BLK, D), lambda i: (i, 0)),
      out_shape=jax.ShapeDtypeStruct((N_ROWS, D), jnp.float32),
  )(x, batch3, a, mb)


@jax.jit
def kernel(x, batch):
  batch = batch.astype(jnp.int32)
  sums, ssq, cnt = _sc_pass1(x, batch)
  a, mb = _tc_combine(sums, ssq, cnt)
  return _tc_pass2(x, batch, a, mb)


# speedup vs baseline: 6.0045x; 1.5543x over previous
import functools

import jax
import jax.numpy as jnp
from jax import lax
from jax.experimental import pallas as pl
from jax.experimental.pallas import tpu as pltpu
from jax.experimental.pallas import tpu_sc as plsc

N_ROWS = 100000
D = 128
NSEG = 64
NLANE = 16
NVEC = D // NLANE

NWORKERS = 32
CHUNK = 320
NCHUNK = -(-N_ROWS // CHUNK)
LAST_BASE = N_ROWS - CHUNK
CHUNKS_PER_WORKER = -(-NCHUNK // NWORKERS)
NGROUP = CHUNK // NLANE


def _sc_pass1_body(x_hbm, batch_hbm, sums_hbm, ssq_hbm, cnt_hbm,
                   xbuf0, xbuf1, bbuf0, bbuf1, acc, ssqacc, cnts,
                   curs, curssq, rp, sx0, sx1, sb0, sb1):
  wid = lax.axis_index("s") * 2 + lax.axis_index("c")
  bufs = ((xbuf0, bbuf0, sx0, sb0), (xbuf1, bbuf1, sx1, sb1))

  def start_dma(k, b):
    c = wid + k * NWORKERS
    base = jnp.minimum(c * CHUNK, LAST_BASE)
    xb, bb, sx, sb = bufs[b]

    @pl.when(c < NCHUNK)
    def _():
      pltpu.async_copy(x_hbm.at[pl.ds(base, CHUNK), :], xb, sx)
      pltpu.async_copy(batch_hbm.at[pl.ds(base, CHUNK)], bb, sb)

  def wait_dma(k, b):
    c = wid + k * NWORKERS
    base = jnp.minimum(c * CHUNK, LAST_BASE)
    xb, bb, sx, sb = bufs[b]
    pltpu.make_async_copy(x_hbm.at[pl.ds(base, CHUNK), :], xb, sx).wait()
    pltpu.make_async_copy(batch_hbm.at[pl.ds(base, CHUNK)], bb, sb).wait()

  start_dma(0, 0)

  zf = jnp.zeros((NLANE,), jnp.float32)

  def _zero(r, carry):
    for j in range(NVEC):
      acc[r, pl.ds(j * NLANE, NLANE)] = zf
    ssqacc[r] = zf
    cnts[r] = jnp.zeros((NLANE,), jnp.int32)
    return carry
  lax.fori_loop(0, NSEG, _zero, 0)
  for j in range(NVEC):
    curs[j] = zf
  curssq[...] = zf
  rp[0] = jnp.int32(-1)
  rp[1] = jnp.int32(0)

  def _flush_vals(prev, s, ssqv, run):
    for j in range(NVEC):
      sl = pl.ds(j * NLANE, NLANE)
      acc[prev, sl] = acc[prev, sl] + s[j]
    ssqacc[prev] = ssqacc[prev] + ssqv
    cnts[prev] = cnts[prev] + jnp.broadcast_to(run, (NLANE,))

  def _flush_cur(prev, run):
    _flush_vals(prev, [curs[j] for j in range(NVEC)], curssq[...], run)
    for j in range(NVEC):
      curs[j] = zf
    curssq[...] = zf

  def process_chunk(c, xbuf, bbuf):
    base = jnp.minimum(c * CHUNK, LAST_BASE)
    skip_g = (c * CHUNK - base) // NLANE

    def group_body(g, carry):
      segv = bbuf[pl.ds(g * NLANE, NLANE)]
      seg0 = segv[0]
      prev = rp[0]
      newseg = seg0 != prev

      @pl.when(newseg & (prev >= 0))
      def _group_flush():
        _flush_cur(prev, rp[1])

      run0 = jnp.where(newseg, 0, rp[1])

      def fast():
        s = [curs[j] for j in range(NVEC)]
        ssqv = curssq[...]
        for i in range(NLANE):
          r = g * NLANE + i
          for j in range(NVEC):
            xj = xbuf[r, pl.ds(j * NLANE, NLANE)]
            s[j] = s[j] + xj
            ssqv = ssqv + xj * xj
        for j in range(NVEC):
          curs[j] = s[j]
        curssq[...] = ssqv
        rp[0] = seg0
        rp[1] = run0 + NLANE

      def slow():
        s = [curs[j] for j in range(NVEC)]
        ssqv = curssq[...]
        run, prev = run0, seg0
        for i in range(NLANE):
          r = g * NLANE + i
          seg = segv[i]
          is_new = seg != prev
          s_now, ssq_now, run_now, prev_now = s, ssqv, run, prev

          @pl.when(is_new)
          def _row_flush():
            _flush_vals(prev_now, s_now, ssq_now, run_now)

          rowsq = zf
          new_s = []
          for j in range(NVEC):
            xj = xbuf[r, pl.ds(j * NLANE, NLANE)]
            rowsq = rowsq + xj * xj
            new_s.append(jnp.where(is_new, xj, s[j] + xj))
          s = new_s
          ssqv = jnp.where(is_new, rowsq, ssqv + rowsq)
          run = jnp.where(is_new, 1, run + 1)
          prev = seg
        for j in range(NVEC):
          curs[j] = s[j]
        curssq[...] = ssqv
        rp[0] = prev
        rp[1] = run

      lax.cond(seg0 == segv[NLANE - 1], fast, slow)
      return carry

    lax.fori_loop(skip_g, NGROUP, group_body, 0)

    prev_end = rp[0]

    @pl.when(prev_end >= 0)
    def _chunk_flush():
      _flush_cur(prev_end, rp[1])
    rp[0] = jnp.int32(-1)
    rp[1] = jnp.int32(0)

  def pair_body(i, carry):
    for b in range(2):
      k = 2 * i + b
      c = wid + k * NWORKERS
      start_dma(k + 1, 1 - b)

      @pl.when(c < NCHUNK)
      def _():
        wait_dma(k, b)
        process_chunk(c, bufs[b][0], bufs[b][1])
    return carry

  lax.fori_loop(0, (CHUNKS_PER_WORKER + 1) // 2, pair_body, 0)

  pltpu.sync_copy(acc, sums_hbm.at[wid])
  pltpu.sync_copy(ssqacc, ssq_hbm.at[wid])
  pltpu.sync_copy(cnts, cnt_hbm.at[wid])


def _sc_pass1(x, batch):
  mesh = plsc.VectorSubcoreMesh(core_axis_name="c", subcore_axis_name="s",
                                num_cores=2, num_subcores=16)
  out_type = (
      jax.ShapeDtypeStruct((NWORKERS, NSEG, D), jnp.float32),
      jax.ShapeDtypeStruct((NWORKERS, NSEG, NLANE), jnp.float32),
      jax.ShapeDtypeStruct((NWORKERS, NSEG, NLANE), jnp.int32),
  )
  scratch = [
      pltpu.VMEM((CHUNK, D), jnp.float32),
      pltpu.VMEM((CHUNK, D), jnp.float32),
      pltpu.VMEM((CHUNK,), jnp.int32),
      pltpu.VMEM((CHUNK,), jnp.int32),
      pltpu.VMEM((NSEG, D), jnp.float32),
      pltpu.VMEM((NSEG, NLANE), jnp.float32),
      pltpu.VMEM((NSEG, NLANE), jnp.int32),
      pltpu.VMEM((NVEC, NLANE), jnp.float32),
      pltpu.VMEM((NLANE,), jnp.float32),
      pltpu.SMEM((2,), jnp.int32),
      pltpu.SemaphoreType.DMA,
      pltpu.SemaphoreType.DMA,
      pltpu.SemaphoreType.DMA,
      pltpu.SemaphoreType.DMA,
  ]
  fn = pl.kernel(_sc_pass1_body, out_type=out_type, mesh=mesh,
                 scratch_types=scratch)
  return fn(x, batch)


def _combine_body(sums_ref, ssq_ref, cnt_ref, a_ref, b_ref):
  tot = jnp.sum(sums_ref[...], axis=0)
  cnt = (jnp.sum(cnt_ref[...], axis=(0, 2)) // NLANE).astype(jnp.float32)
  cntc = jnp.maximum(cnt, 1.0)
  mean = tot / cntc[:, None]
  ssq_tot = jnp.sum(ssq_ref[...], axis=(0, 2))
  sq_mean = jnp.maximum(ssq_tot / cntc - jnp.sum(mean * mean, axis=1), 0.0)
  rstd = jnp.where(sq_mean > 0, lax.rsqrt(sq_mean), 0.0)
  a_ref[...] = jnp.broadcast_to(rstd[:, None], (NSEG, D))
  b_ref[...] = mean * rstd[:, None]


def _tc_combine(sums, ssq, cnt):
  return pl.pallas_call(
      _combine_body,
      out_shape=(
          jax.ShapeDtypeStruct((NSEG, D), jnp.float32),
          jax.ShapeDtypeStruct((NSEG, D), jnp.float32),
      ),
  )(sums, ssq, cnt)


BLK = 2000
NBLK = N_ROWS // BLK


def _pass2_body(x_ref, b_ref, a_ref, mb_ref, o_ref):
  b = b_ref[0, 0, :]
  oh = (b[:, None] == lax.broadcasted_iota(jnp.int32, (BLK, NSEG), 1))
  oh = oh.astype(jnp.float32)
  rstd = jnp.dot(oh, a_ref[...], preferred_element_type=jnp.float32)
  mscaled = jnp.dot(oh, mb_ref[...], preferred_element_type=jnp.float32)
  o_ref[...] = x_ref[...] * rstd - mscaled


def _tc_pass2(x, batch, a, mb):
  batch3 = batch.reshape(NBLK, 1, BLK)
  return pl.pallas_call(
      _pass2_body,
      grid=(NBLK,),
      in_specs=[
          pl.BlockSpec((BLK, D), lambda i: (i, 0)),
          pl.BlockSpec((1, 1, BLK), lambda i: (i, 0, 0)),
          pl.BlockSpec((NSEG, D), lambda i: (0, 0)),
          pl.BlockSpec((NSEG, D), lambda i: (0, 0)),
      ],
      out_specs=pl.BlockSpec((BLK, D), lambda i: (i, 0)),
      out_shape=jax.ShapeDtypeStruct((N_ROWS, D), jnp.float32),
  )(x, batch3, a, mb)


@jax.jit
def kernel(x, batch):
  batch = batch.astype(jnp.int32)
  sums, ssq, cnt = _sc_pass1(x, batch)
  a, mb = _tc_combine(sums, ssq, cnt)
  return _tc_pass2(x, batch, a, mb)


# speedup vs baseline: 9.3477x; 1.5568x over previous
import functools

import jax
import jax.numpy as jnp
from jax import lax
from jax.experimental import pallas as pl
from jax.experimental.pallas import tpu as pltpu
from jax.experimental.pallas import tpu_sc as plsc

N_ROWS = 100000
D = 128
NSEG = 64
NLANE = 16
NVEC = D // NLANE

NWORKERS = 32
CHUNK = 320

TCBLK1 = 4000
TC_ROWS = TCBLK1 * 14
SC_R0 = TC_ROWS
SC_ROWS = N_ROWS - SC_R0
NCHUNK = -(-SC_ROWS // CHUNK)
LAST_BASE = N_ROWS - CHUNK
CHUNKS_PER_WORKER = -(-NCHUNK // NWORKERS)
NGROUP = CHUNK // NLANE


def _sc_pass1_body(x_hbm, batch_hbm, sums_hbm, ssq_hbm, cnt_hbm,
                   xbuf0, xbuf1, bbuf0, bbuf1, acc, ssqacc, cnts,
                   curs, curssq, rp, sx0, sx1, sb0, sb1):
  wid = lax.axis_index("s") * 2 + lax.axis_index("c")
  bufs = ((xbuf0, bbuf0, sx0, sb0), (xbuf1, bbuf1, sx1, sb1))

  def start_dma(k, b):
    c = wid + k * NWORKERS
    base = jnp.minimum(SC_R0 + c * CHUNK, LAST_BASE)
    xb, bb, sx, sb = bufs[b]

    @pl.when(c < NCHUNK)
    def _():
      pltpu.async_copy(x_hbm.at[pl.ds(base, CHUNK), :], xb, sx)
      pltpu.async_copy(batch_hbm.at[pl.ds(base, CHUNK)], bb, sb)

  def wait_dma(k, b):
    c = wid + k * NWORKERS
    base = jnp.minimum(SC_R0 + c * CHUNK, LAST_BASE)
    xb, bb, sx, sb = bufs[b]
    pltpu.make_async_copy(x_hbm.at[pl.ds(base, CHUNK), :], xb, sx).wait()
    pltpu.make_async_copy(batch_hbm.at[pl.ds(base, CHUNK)], bb, sb).wait()

  start_dma(0, 0)

  zf = jnp.zeros((NLANE,), jnp.float32)

  def _zero(r, carry):
    for j in range(NVEC):
      acc[r, pl.ds(j * NLANE, NLANE)] = zf
    ssqacc[r] = zf
    cnts[r] = jnp.zeros((NLANE,), jnp.int32)
    return carry
  lax.fori_loop(0, NSEG, _zero, 0)
  for j in range(NVEC):
    curs[j] = zf
    curssq[j] = zf
  rp[0] = jnp.int32(-1)
  rp[1] = jnp.int32(0)

  def _sum8(v):
    v = list(v)
    while len(v) > 1:
      v = [v[k] + v[k + 1] for k in range(0, len(v), 2)]
    return v[0]

  def _flush_vals(prev, s, ssq8, run):
    for j in range(NVEC):
      sl = pl.ds(j * NLANE, NLANE)
      acc[prev, sl] = acc[prev, sl] + s[j]
    ssqacc[prev] = ssqacc[prev] + _sum8(ssq8)
    cnts[prev] = cnts[prev] + jnp.broadcast_to(run, (NLANE,))

  def _flush_cur(prev, run):
    _flush_vals(prev, [curs[j] for j in range(NVEC)],
                [curssq[j] for j in range(NVEC)], run)
    for j in range(NVEC):
      curs[j] = zf
      curssq[j] = zf

  def process_chunk(c, xbuf, bbuf):
    base = jnp.minimum(SC_R0 + c * CHUNK, LAST_BASE)
    skip_g = (SC_R0 + c * CHUNK - base) // NLANE

    def group_body(g, carry):
      segv = bbuf[pl.ds(g * NLANE, NLANE)]
      seg0 = segv[0]
      prev = rp[0]
      newseg = seg0 != prev

      @pl.when(newseg & (prev >= 0))
      def _group_flush():
        _flush_cur(prev, rp[1])

      run0 = jnp.where(newseg, 0, rp[1])

      def fast():
        s = [curs[j] for j in range(NVEC)]
        q = [curssq[j] for j in range(NVEC)]
        for i in range(NLANE):
          r = g * NLANE + i
          for j in range(NVEC):
            xj = xbuf[r, pl.ds(j * NLANE, NLANE)]
            s[j] = s[j] + xj
            q[j] = q[j] + xj * xj
        for j in range(NVEC):
          curs[j] = s[j]
          curssq[j] = q[j]
        rp[0] = seg0
        rp[1] = run0 + NLANE

      def slow():
        s = [curs[j] for j in range(NVEC)]
        q = [curssq[j] for j in range(NVEC)]
        run, prev = run0, seg0
        for i in range(NLANE):
          r = g * NLANE + i
          seg = segv[i]
          is_new = seg != prev
          s_now, q_now, run_now, prev_now = s, q, run, prev

          @pl.when(is_new)
          def _row_flush():
            _flush_vals(prev_now, s_now, q_now, run_now)

          new_s, new_q = [], []
          for j in range(NVEC):
            xj = xbuf[r, pl.ds(j * NLANE, NLANE)]
            new_s.append(jnp.where(is_new, xj, s[j] + xj))
            new_q.append(jnp.where(is_new, xj * xj, q[j] + xj * xj))
          s, q = new_s, new_q
          run = jnp.where(is_new, 1, run + 1)
          prev = seg
        for j in range(NVEC):
          curs[j] = s[j]
          curssq[j] = q[j]
        rp[0] = prev
        rp[1] = run

      lax.cond(seg0 == segv[NLANE - 1], fast, slow)
      return carry

    lax.fori_loop(skip_g, NGROUP, group_body, 0)

    prev_end = rp[0]

    @pl.when(prev_end >= 0)
    def _chunk_flush():
      _flush_cur(prev_end, rp[1])
    rp[0] = jnp.int32(-1)
    rp[1] = jnp.int32(0)

  def pair_body(i, carry):
    for b in range(2):
      k = 2 * i + b
      c = wid + k * NWORKERS
      start_dma(k + 1, 1 - b)

      @pl.when(c < NCHUNK)
      def _():
        wait_dma(k, b)
        process_chunk(c, bufs[b][0], bufs[b][1])
    return carry

  lax.fori_loop(0, (CHUNKS_PER_WORKER + 1) // 2, pair_body, 0)

  pltpu.sync_copy(acc, sums_hbm.at[wid])
  pltpu.sync_copy(ssqacc, ssq_hbm.at[wid])
  pltpu.sync_copy(cnts, cnt_hbm.at[wid])


def _sc_pass1(x, batch):
  mesh = plsc.VectorSubcoreMesh(core_axis_name="c", subcore_axis_name="s",
                                num_cores=2, num_subcores=16)
  out_type = (
      jax.ShapeDtypeStruct((NWORKERS, NSEG, D), jnp.float32),
      jax.ShapeDtypeStruct((NWORKERS, NSEG, NLANE), jnp.float32),
      jax.ShapeDtypeStruct((NWORKERS, NSEG, NLANE), jnp.int32),
  )
  scratch = [
      pltpu.VMEM((CHUNK, D), jnp.float32),
      pltpu.VMEM((CHUNK, D), jnp.float32),
      pltpu.VMEM((CHUNK,), jnp.int32),
      pltpu.VMEM((CHUNK,), jnp.int32),
      pltpu.VMEM((NSEG, D), jnp.float32),
      pltpu.VMEM((NSEG, NLANE), jnp.float32),
      pltpu.VMEM((NSEG, NLANE), jnp.int32),
      pltpu.VMEM((NVEC, NLANE), jnp.float32),
      pltpu.VMEM((NVEC, NLANE), jnp.float32),
      pltpu.SMEM((2,), jnp.int32),
      pltpu.SemaphoreType.DMA,
      pltpu.SemaphoreType.DMA,
      pltpu.SemaphoreType.DMA,
      pltpu.SemaphoreType.DMA,
  ]
  fn = pl.kernel(_sc_pass1_body, out_type=out_type, mesh=mesh,
                 scratch_types=scratch)
  return fn(x, batch)


def _tc_pass1_body(x_ref, b_ref, sums_ref, cnt_ref, ssq_ref):
  i = pl.program_id(0)

  @pl.when(i == 0)
  def _init():
    sums_ref[...] = jnp.zeros((NSEG, D), jnp.float32)
    cnt_ref[...] = jnp.zeros((1, NSEG), jnp.float32)
    ssq_ref[...] = jnp.zeros((NSEG, D), jnp.float32)

  b = b_ref[0, 0, :]
  oh = (b[:, None] == lax.broadcasted_iota(jnp.int32, (TCBLK1, NSEG), 1))
  oh = oh.astype(jnp.float32)
  x = x_ref[...]
  dn = (((0,), (0,)), ((), ()))
  sums_ref[...] += lax.dot_general(oh, x, dn,
                                   preferred_element_type=jnp.float32)
  ssq_ref[...] += lax.dot_general(oh, x * x, dn,
                                  preferred_element_type=jnp.float32)
  cnt_ref[...] += jnp.sum(oh, axis=0)[None, :]


def _tc_pass1(x, batch3):
  nblk = TC_ROWS // TCBLK1
  return pl.pallas_call(
      _tc_pass1_body,
      grid=(nblk,),
      in_specs=[
          pl.BlockSpec((TCBLK1, D), lambda i: (i, 0)),
          pl.BlockSpec((1, 1, TCBLK1), lambda i: (i, 0, 0)),
      ],
      out_specs=(
          pl.BlockSpec((NSEG, D), lambda i: (0, 0)),
          pl.BlockSpec((1, NSEG), lambda i: (0, 0)),
          pl.BlockSpec((NSEG, D), lambda i: (0, 0)),
      ),
      out_shape=(
          jax.ShapeDtypeStruct((NSEG, D), jnp.float32),
          jax.ShapeDtypeStruct((1, NSEG), jnp.float32),
          jax.ShapeDtypeStruct((NSEG, D), jnp.float32),
      ),
  )(x, batch3)


BLK = 4000
NBLK = N_ROWS // BLK


def _pass2_body(x_ref, b_ref, ss_ref, sq_ref, sc_ref, tcs_ref, tcc_ref,
                tcq_ref, o_ref, a_s, mb_s):
  i = pl.program_id(0)

  @pl.when(i == 0)
  def _combine():
    tot = jnp.sum(ss_ref[...], axis=0) + tcs_ref[...]
    cnt = ((jnp.sum(sc_ref[...], axis=(0, 2)) // NLANE).astype(jnp.float32)
           + tcc_ref[0, :])
    cntc = jnp.maximum(cnt, 1.0)
    mean = tot / cntc[:, None]
    ssq_tot = (jnp.sum(sq_ref[...], axis=(0, 2))
               + jnp.sum(tcq_ref[...], axis=1))
    sq_mean = jnp.maximum(ssq_tot / cntc - jnp.sum(mean * mean, axis=1), 0.0)
    rstd = jnp.where(sq_mean > 0, lax.rsqrt(sq_mean), 0.0)
    a_s[...] = jnp.broadcast_to(rstd[:, None], (NSEG, D))
    mb_s[...] = mean * rstd[:, None]

  b = b_ref[0, 0, :]
  oh = (b[:, None] == lax.broadcasted_iota(jnp.int32, (BLK, NSEG), 1))
  oh = oh.astype(jnp.float32)
  rstd = jnp.dot(oh, a_s[...], preferred_element_type=jnp.float32)
  mscaled = jnp.dot(oh, mb_s[...], preferred_element_type=jnp.float32)
  o_ref[...] = x_ref[...] * rstd - mscaled


def _tc_pass2(x, batch3, sums, ssq, cnt, tcs, tcc, tcq, blk0, nblk):
  full = lambda shape: pl.BlockSpec(shape, lambda i: tuple(0 for _ in shape))
  return pl.pallas_call(
      _pass2_body,
      grid=(nblk,),
      in_specs=[
          pl.BlockSpec((BLK, D), lambda i: (i + blk0, 0)),
          pl.BlockSpec((1, 1, BLK), lambda i: (i + blk0, 0, 0)),
          full((NWORKERS, NSEG, D)),
          full((NWORKERS, NSEG, NLANE)),
          full((NWORKERS, NSEG, NLANE)),
          full((NSEG, D)),
          full((1, NSEG)),
          full((NSEG, D)),
      ],
      out_specs=pl.BlockSpec((BLK, D), lambda i: (i, 0)),
      out_shape=jax.ShapeDtypeStruct((nblk * BLK, D), jnp.float32),
      scratch_shapes=[
          pltpu.VMEM((NSEG, D), jnp.float32),
          pltpu.VMEM((NSEG, D), jnp.float32),
      ],
  )(x, batch3, sums, ssq, cnt, tcs, tcc, tcq)


@jax.jit
def kernel(x, batch):
  batch = batch.astype(jnp.int32)
  batch3 = batch.reshape(NBLK, 1, BLK)
  sums, ssq, cnt = _sc_pass1(x, batch)
  tcs, tcc, tcq = _tc_pass1(x, batch3)
  return _tc_pass2(x, batch3, sums, ssq, cnt, tcs, tcc, tcq, 0, NBLK)
